# baseline (speedup 1.0000x reference)
import functools

import jax
import jax.numpy as jnp
from jax import lax
from jax.experimental import pallas as pl
from jax.experimental.pallas import tpu as pltpu
from jax.experimental.pallas import tpu_sc as plsc

CLASSES = 128
SUPER = 8
CF = CLASSES // SUPER
BW = 2048
NTOK = 4 * 8192
NWORK = 32
NSLAB = 2
NTOK_S = NTOK // NSLAB
CHUNK = NTOK_S // NWORK

_F32 = jnp.float32


def _lrelu(v):
    return jnp.maximum(v, 0.01 * v)


def _mm(w, v):
    return jax.lax.dot_general(w, v, (((1,), (0,)), ((), ())),
                               preferred_element_type=_F32)


def _tc_body(x_ref, cl1_w_ref, cl1_b_ref, cl2_w_ref, cl2_b_ref, cl3_w_ref,
             cl3_b_ref, reg1_w_ref, reg1_b_ref, w2r_ref, w2h_ref, b2t_ref,
             x32_ref, inds_ref, mask_ref):
    x = x_ref[0, :, 0, :]

    h1 = _lrelu(_mm(cl1_w_ref[...], x) + cl1_b_ref[...].reshape(128, 1))
    h2 = _lrelu(_mm(cl2_w_ref[...], h1) + cl2_b_ref[...].reshape(128, 1))
    lg = _mm(cl3_w_ref[...], h2) + cl3_b_ref[...].reshape(CLASSES + 1, 1)
    mask_ref[0, 0, 0, :] = _lrelu(lg[CLASSES, :])

    cls = lg[0:CLASSES, :]
    m = jnp.max(cls, axis=0, keepdims=True)
    row_iota = jax.lax.broadcasted_iota(jnp.int32, (CLASSES, BW), 0)
    inds = jnp.min(jnp.where(cls == m, row_iota, CLASSES), axis=0,
                   keepdims=True)
    inds_ref[...] = inds

    r1 = _lrelu(_mm(reg1_w_ref[...], x) + reg1_b_ref[...].reshape(128, 1))
    y = (_mm(w2r_ref[...], r1) +
         _mm(w2h_ref[...], h1))

    s = inds // CF
    oh8 = (jax.lax.broadcasted_iota(jnp.int32, (SUPER, BW), 0)
           == s).astype(_F32)
    b32 = _mm(b2t_ref[...], oh8)
    x32 = y[0:32, :]
    for e in range(1, SUPER):
        x32 = jnp.where(s == e, y[e * 32:(e + 1) * 32, :], x32)
    x32_ref[...] = _lrelu(x32 + b32)


def _sc_body(x32_hbm, inds_hbm, w3bt_hbm, out_hbm,
             idx_v, x32_v, w3t_v, out_v):
    wid = lax.axis_index("s") * 2 + lax.axis_index("c")
    base = wid * CHUNK
    pltpu.sync_copy(inds_hbm.at[0, pl.ds(base, CHUNK)], idx_v)
    pltpu.sync_copy(x32_hbm.at[:, pl.ds(base, CHUNK)], x32_v)
    pltpu.sync_copy(w3bt_hbm, w3t_v)

    def group(g, carry):
        b16 = g * 16
        idx16 = idx_v[pl.ds(b16, 16)]
        acc = jnp.zeros((16,), _F32)
        for j in range(32):
            xj = x32_v[j, pl.ds(b16, 16)]
            wj = plsc.load_gather(w3t_v, [idx16 + (j * CLASSES)])
            acc = acc + xj * wj
        bias = plsc.load_gather(w3t_v, [idx16 + (32 * CLASSES)])
        out_v[pl.ds(b16, 16)] = ((idx16.astype(_F32) + acc + bias) *
                                 (1.0 / float(CLASSES)))
        return carry

    lax.fori_loop(0, CHUNK // 16, group, 0)
    pltpu.sync_copy(out_v, out_hbm.at[pl.ds(base, CHUNK)])


def _tc_slab(x_slab, cl1_w, cl1_b, cl2_w, cl2_b, cl3_w, cl3_b,
             reg1_w, reg1_b, w2r, w2h, b2t):
    B, C, H, W = x_slab.shape
    grid = (B, W // BW)
    nj = W // BW
    wspec = lambda shape: pl.BlockSpec(shape, lambda b, j: (0,) * len(shape))
    out_shapes = (
        jax.ShapeDtypeStruct((32, NTOK_S), jnp.float32),
        jax.ShapeDtypeStruct((1, NTOK_S), jnp.int32),
        jax.ShapeDtypeStruct((B, 1, 1, W), jnp.float32),
    )
    return pl.pallas_call(
        _tc_body,
        grid=grid,
        in_specs=[
            pl.BlockSpec((1, C, 1, BW), lambda b, j: (b, 0, 0, j)),
            wspec((128, 128)), wspec((128,)),
            wspec((128, 128)), wspec((128,)),
            wspec((CLASSES + 1, 128)), wspec((CLASSES + 1,)),
            wspec((128, 128)), wspec((128,)),
            wspec((256, 128)), wspec((256, 128)),
            wspec((32, SUPER)),
        ],
        out_specs=(
            pl.BlockSpec((32, BW), lambda b, j: (0, b * nj + j)),
            pl.BlockSpec((1, BW), lambda b, j: (0, b * nj + j)),
            pl.BlockSpec((1, 1, 1, BW), lambda b, j: (b, 0, 0, j)),
        ),
        out_shape=out_shapes,
    )(x_slab, cl1_w, cl1_b, cl2_w, cl2_b, cl3_w, cl3_b,
      reg1_w, reg1_b, w2r, w2h, b2t)


@jax.jit
def _run(x_in, cl1_w, cl1_b, cl2_w, cl2_b, cl3_w, cl3_b,
         reg1_w, reg1_b, w2r, w2h, b2t, w3b):
    B, C, H, W = x_in.shape
    ws = W // NSLAB

    mesh = plsc.VectorSubcoreMesh(core_axis_name="c", subcore_axis_name="s")
    sc = functools.partial(
        pl.kernel, mesh=mesh,
        compiler_params=pltpu.CompilerParams(needs_layout_passes=False),
        out_type=jax.ShapeDtypeStruct((NTOK_S,), jnp.float32),
        scratch_types=[
            pltpu.VMEM((CHUNK,), jnp.int32),
            pltpu.VMEM((32, CHUNK), jnp.float32),
            pltpu.VMEM((33 * CLASSES,), jnp.float32),
            pltpu.VMEM((CHUNK,), jnp.float32),
        ],
    )(_sc_body)

    tc_outs = []
    for i in range(NSLAB):
        x_slab = jax.lax.slice_in_dim(x_in, i * ws, (i + 1) * ws, axis=3)
        tc_outs.append(_tc_slab(x_slab, cl1_w, cl1_b, cl2_w, cl2_b, cl3_w,
                                cl3_b, reg1_w, reg1_b, w2r, w2h, b2t))
    xr_slabs = [sc(x32, inds, w3b).reshape(B, 1, 1, ws)
                for (x32, inds, _) in tc_outs]
    x_real = jnp.concatenate(xr_slabs, axis=3)
    mask = jnp.concatenate([m for (_, _, m) in tc_outs], axis=3)
    return x_real, mask


def kernel(x_in, cl1_w, cl1_b, cl2_w, cl2_b, cl3_w, cl3_b,
           reg1_w, reg1_b, reg2_w, reg2_b, reg3_w, reg3_b):
    w2all = jnp.transpose(reg2_w, (0, 2, 1)).reshape(SUPER * 32, 256)
    w2r = w2all[:, 0:128]
    w2h = w2all[:, 128:256]
    b2t = reg2_b.T
    w3b = jnp.concatenate(
        [reg3_w[:, :, 0].T, reg3_b[:, 0].reshape(1, CLASSES)],
        axis=0).reshape(-1)
    x_real, mask = _run(x_in, cl1_w, cl1_b, cl2_w, cl2_b, cl3_w, cl3_b,
                        reg1_w, reg1_b, w2r, w2h, b2t, w3b)
    return (x_real, mask)


# speedup vs baseline: 1.1312x; 1.1312x over previous
import functools

import jax
import jax.numpy as jnp
from jax import lax
from jax.experimental import pallas as pl
from jax.experimental.pallas import tpu as pltpu
from jax.experimental.pallas import tpu_sc as plsc

CLASSES = 128
SUPER = 8
CF = CLASSES // SUPER
BW = 2048
NTOK = 4 * 8192
NWORK = 32
NSLAB = 1
NTOK_S = NTOK // NSLAB
CHUNK = NTOK_S // NWORK

_F32 = jnp.float32


def _lrelu(v):
    return jnp.maximum(v, 0.01 * v)


def _mm(w, v):
    return jax.lax.dot_general(w, v, (((1,), (0,)), ((), ())),
                               preferred_element_type=_F32)


def _tc_body(x_ref, cl1_w_ref, cl1_b_ref, cl2_w_ref, cl2_b_ref, cl3_w_ref,
             cl3_b_ref, reg1_w_ref, reg1_b_ref, w2r_ref, w2h_ref, b2t_ref,
             x32_ref, inds_ref, mask_ref):
    x = x_ref[0, :, 0, :]

    h1 = _lrelu(_mm(cl1_w_ref[...], x) + cl1_b_ref[...].reshape(128, 1))
    h2 = _lrelu(_mm(cl2_w_ref[...], h1) + cl2_b_ref[...].reshape(128, 1))
    lg = _mm(cl3_w_ref[...], h2) + cl3_b_ref[...].reshape(CLASSES + 1, 1)
    mask_ref[0, 0, 0, :] = _lrelu(lg[CLASSES, :])

    cls = lg[0:CLASSES, :]
    m = jnp.max(cls, axis=0, keepdims=True)
    row_iota = jax.lax.broadcasted_iota(jnp.int32, (CLASSES, BW), 0)
    inds = jnp.min(jnp.where(cls == m, row_iota, CLASSES), axis=0,
                   keepdims=True)
    inds_ref[...] = inds

    r1 = _lrelu(_mm(reg1_w_ref[...], x) + reg1_b_ref[...].reshape(128, 1))
    y = (_mm(w2r_ref[...], r1) +
         _mm(w2h_ref[...], h1))

    s = inds // CF
    oh8 = (jax.lax.broadcasted_iota(jnp.int32, (SUPER, BW), 0)
           == s).astype(_F32)
    b32 = _mm(b2t_ref[...], oh8)
    x32 = y[0:32, :]
    for e in range(1, SUPER):
        x32 = jnp.where(s == e, y[e * 32:(e + 1) * 32, :], x32)
    x32_ref[...] = _lrelu(x32 + b32)


def _sc_body(x32_hbm, inds_hbm, w3bt_hbm, out_hbm,
             idx_v, x32_v, w3t_v, out_v):
    wid = lax.axis_index("s") * 2 + lax.axis_index("c")
    base = wid * CHUNK
    pltpu.sync_copy(inds_hbm.at[0, pl.ds(base, CHUNK)], idx_v)
    pltpu.sync_copy(x32_hbm.at[:, pl.ds(base, CHUNK)], x32_v)
    pltpu.sync_copy(w3bt_hbm, w3t_v)

    @plsc.parallel_loop(0, CHUNK // 16, unroll=4)
    def group(g):
        b16 = g * 16
        idx16 = idx_v[pl.ds(b16, 16)]
        acc = plsc.load_gather(w3t_v, [idx16 + (32 * CLASSES)])
        for j in range(32):
            xj = x32_v[j, pl.ds(b16, 16)]
            wj = plsc.load_gather(w3t_v, [idx16 + (j * CLASSES)])
            acc = acc + xj * wj
        out_v[pl.ds(b16, 16)] = ((idx16.astype(_F32) + acc) *
                                 (1.0 / float(CLASSES)))
    pltpu.sync_copy(out_v, out_hbm.at[pl.ds(base, CHUNK)])


def _tc_slab(x_slab, cl1_w, cl1_b, cl2_w, cl2_b, cl3_w, cl3_b,
             reg1_w, reg1_b, w2r, w2h, b2t):
    B, C, H, W = x_slab.shape
    grid = (B, W // BW)
    nj = W // BW
    wspec = lambda shape: pl.BlockSpec(shape, lambda b, j: (0,) * len(shape))
    out_shapes = (
        jax.ShapeDtypeStruct((32, NTOK_S), jnp.float32),
        jax.ShapeDtypeStruct((1, NTOK_S), jnp.int32),
        jax.ShapeDtypeStruct((B, 1, 1, W), jnp.float32),
    )
    return pl.pallas_call(
        _tc_body,
        grid=grid,
        in_specs=[
            pl.BlockSpec((1, C, 1, BW), lambda b, j: (b, 0, 0, j)),
            wspec((128, 128)), wspec((128,)),
            wspec((128, 128)), wspec((128,)),
            wspec((CLASSES + 1, 128)), wspec((CLASSES + 1,)),
            wspec((128, 128)), wspec((128,)),
            wspec((256, 128)), wspec((256, 128)),
            wspec((32, SUPER)),
        ],
        out_specs=(
            pl.BlockSpec((32, BW), lambda b, j: (0, b * nj + j)),
            pl.BlockSpec((1, BW), lambda b, j: (0, b * nj + j)),
            pl.BlockSpec((1, 1, 1, BW), lambda b, j: (b, 0, 0, j)),
        ),
        out_shape=out_shapes,
    )(x_slab, cl1_w, cl1_b, cl2_w, cl2_b, cl3_w, cl3_b,
      reg1_w, reg1_b, w2r, w2h, b2t)


@jax.jit
def _run(x_in, cl1_w, cl1_b, cl2_w, cl2_b, cl3_w, cl3_b,
         reg1_w, reg1_b, w2r, w2h, b2t, w3b):
    B, C, H, W = x_in.shape
    ws = W // NSLAB

    mesh = plsc.VectorSubcoreMesh(core_axis_name="c", subcore_axis_name="s")
    sc = functools.partial(
        pl.kernel, mesh=mesh,
        compiler_params=pltpu.CompilerParams(needs_layout_passes=False),
        out_type=jax.ShapeDtypeStruct((NTOK_S,), jnp.float32),
        scratch_types=[
            pltpu.VMEM((CHUNK,), jnp.int32),
            pltpu.VMEM((32, CHUNK), jnp.float32),
            pltpu.VMEM((33 * CLASSES,), jnp.float32),
            pltpu.VMEM((CHUNK,), jnp.float32),
        ],
    )(_sc_body)

    tc_outs = []
    for i in range(NSLAB):
        x_slab = jax.lax.slice_in_dim(x_in, i * ws, (i + 1) * ws, axis=3)
        tc_outs.append(_tc_slab(x_slab, cl1_w, cl1_b, cl2_w, cl2_b, cl3_w,
                                cl3_b, reg1_w, reg1_b, w2r, w2h, b2t))
    xr_slabs = [sc(x32, inds, w3b).reshape(B, 1, 1, ws)
                for (x32, inds, _) in tc_outs]
    x_real = jnp.concatenate(xr_slabs, axis=3)
    mask = jnp.concatenate([m for (_, _, m) in tc_outs], axis=3)
    return x_real, mask


def kernel(x_in, cl1_w, cl1_b, cl2_w, cl2_b, cl3_w, cl3_b,
           reg1_w, reg1_b, reg2_w, reg2_b, reg3_w, reg3_b):
    w2all = jnp.transpose(reg2_w, (0, 2, 1)).reshape(SUPER * 32, 256)
    w2r = w2all[:, 0:128]
    w2h = w2all[:, 128:256]
    b2t = reg2_b.T
    w3b = jnp.concatenate(
        [reg3_w[:, :, 0].T, reg3_b[:, 0].reshape(1, CLASSES)],
        axis=0).reshape(-1)
    x_real, mask = _run(x_in, cl1_w, cl1_b, cl2_w, cl2_b, cl3_w, cl3_b,
                        reg1_w, reg1_b, w2r, w2h, b2t, w3b)
    return (x_real, mask)


# speedup vs baseline: 1.1748x; 1.0386x over previous
import functools

import jax
import jax.numpy as jnp
from jax import lax
from jax.experimental import pallas as pl
from jax.experimental.pallas import tpu as pltpu
from jax.experimental.pallas import tpu_sc as plsc

CLASSES = 128
SUPER = 8
CF = CLASSES // SUPER
BW = 2048
NTOK = 4 * 8192
NWORK = 32
NSLAB = 1
NTOK_S = NTOK // NSLAB
CHUNK = NTOK_S // NWORK

_F32 = jnp.float32


def _lrelu(v):
    return jnp.maximum(v, 0.01 * v)


def _mm(w, v):
    return jax.lax.dot_general(w, v, (((1,), (0,)), ((), ())),
                               preferred_element_type=_F32)


def _tc_body(x_ref, cl1_w_ref, cl1_b_ref, cl2_w_ref, cl2_b_ref, cl3_w_ref,
             cl3_b_ref, reg1_w_ref, reg1_b_ref, w2r_ref, w2h_ref, b2t_ref,
             x32_ref, inds_ref, mask_ref):
    x = x_ref[0, :, 0, :]

    h1 = _lrelu(_mm(cl1_w_ref[...], x) + cl1_b_ref[...].reshape(128, 1))
    h2 = _lrelu(_mm(cl2_w_ref[...], h1) + cl2_b_ref[...].reshape(128, 1))
    lg = _mm(cl3_w_ref[...], h2) + cl3_b_ref[...].reshape(CLASSES + 1, 1)
    mask_ref[0, 0, 0, :] = _lrelu(lg[CLASSES, :])

    cls = lg[0:CLASSES, :]
    m = jnp.max(cls, axis=0, keepdims=True)
    row_iota = jax.lax.broadcasted_iota(jnp.int32, (CLASSES, BW), 0)
    inds = jnp.min(jnp.where(cls == m, row_iota, CLASSES), axis=0,
                   keepdims=True)
    inds_ref[...] = inds

    r1 = _lrelu(_mm(reg1_w_ref[...], x) + reg1_b_ref[...].reshape(128, 1))
    y = (_mm(w2r_ref[...], r1) +
         _mm(w2h_ref[...], h1))

    s = inds // CF
    oh8 = (jax.lax.broadcasted_iota(jnp.int32, (SUPER, BW), 0)
           == s).astype(_F32)
    b32 = _mm(b2t_ref[...], oh8)
    x32 = y[0:32, :]
    for e in range(1, SUPER):
        x32 = jnp.where(s == e, y[e * 32:(e + 1) * 32, :], x32)
    x32_ref[...] = _lrelu(x32 + b32)


def _sc_body(x32_hbm, inds_hbm, w3bt_hbm, out_hbm,
             idx_v, x32_v, w3t_v, out_v):
    wid = lax.axis_index("s") * 2 + lax.axis_index("c")
    base = wid * CHUNK
    pltpu.sync_copy(inds_hbm.at[0, pl.ds(base, CHUNK)], idx_v)
    pltpu.sync_copy(x32_hbm.at[:, pl.ds(base, CHUNK)], x32_v)
    pltpu.sync_copy(w3bt_hbm, w3t_v)

    @plsc.parallel_loop(0, CHUNK // 16, unroll=2)
    def group(g):
        b16 = g * 16
        idx16 = idx_v[pl.ds(b16, 16)]
        acc = plsc.load_gather(w3t_v, [idx16 + (32 * CLASSES)])
        for j in range(32):
            xj = x32_v[j, pl.ds(b16, 16)]
            wj = plsc.load_gather(w3t_v, [idx16 + (j * CLASSES)])
            acc = acc + xj * wj
        out_v[pl.ds(b16, 16)] = ((idx16.astype(_F32) + acc) *
                                 (1.0 / float(CLASSES)))
    pltpu.sync_copy(out_v, out_hbm.at[pl.ds(base, CHUNK)])


def _tc_slab(x_slab, cl1_w, cl1_b, cl2_w, cl2_b, cl3_w, cl3_b,
             reg1_w, reg1_b, w2r, w2h, b2t):
    B, C, H, W = x_slab.shape
    grid = (B, W // BW)
    nj = W // BW
    wspec = lambda shape: pl.BlockSpec(shape, lambda b, j: (0,) * len(shape))
    out_shapes = (
        jax.ShapeDtypeStruct((32, NTOK_S), jnp.float32),
        jax.ShapeDtypeStruct((1, NTOK_S), jnp.int32),
        jax.ShapeDtypeStruct((B, 1, 1, W), jnp.float32),
    )
    return pl.pallas_call(
        _tc_body,
        grid=grid,
        in_specs=[
            pl.BlockSpec((1, C, 1, BW), lambda b, j: (b, 0, 0, j)),
            wspec((128, 128)), wspec((128,)),
            wspec((128, 128)), wspec((128,)),
            wspec((CLASSES + 1, 128)), wspec((CLASSES + 1,)),
            wspec((128, 128)), wspec((128,)),
            wspec((256, 128)), wspec((256, 128)),
            wspec((32, SUPER)),
        ],
        out_specs=(
            pl.BlockSpec((32, BW), lambda b, j: (0, b * nj + j)),
            pl.BlockSpec((1, BW), lambda b, j: (0, b * nj + j)),
            pl.BlockSpec((1, 1, 1, BW), lambda b, j: (b, 0, 0, j)),
        ),
        out_shape=out_shapes,
    )(x_slab, cl1_w, cl1_b, cl2_w, cl2_b, cl3_w, cl3_b,
      reg1_w, reg1_b, w2r, w2h, b2t)


@jax.jit
def _run(x_in, cl1_w, cl1_b, cl2_w, cl2_b, cl3_w, cl3_b,
         reg1_w, reg1_b, w2r, w2h, b2t, w3b):
    B, C, H, W = x_in.shape
    ws = W // NSLAB

    mesh = plsc.VectorSubcoreMesh(core_axis_name="c", subcore_axis_name="s")
    sc = functools.partial(
        pl.kernel, mesh=mesh,
        compiler_params=pltpu.CompilerParams(needs_layout_passes=False),
        out_type=jax.ShapeDtypeStruct((NTOK_S,), jnp.float32),
        scratch_types=[
            pltpu.VMEM((CHUNK,), jnp.int32),
            pltpu.VMEM((32, CHUNK), jnp.float32),
            pltpu.VMEM((33 * CLASSES,), jnp.float32),
            pltpu.VMEM((CHUNK,), jnp.float32),
        ],
    )(_sc_body)

    tc_outs = []
    for i in range(NSLAB):
        x_slab = jax.lax.slice_in_dim(x_in, i * ws, (i + 1) * ws, axis=3)
        tc_outs.append(_tc_slab(x_slab, cl1_w, cl1_b, cl2_w, cl2_b, cl3_w,
                                cl3_b, reg1_w, reg1_b, w2r, w2h, b2t))
    xr_slabs = [sc(x32, inds, w3b).reshape(B, 1, 1, ws)
                for (x32, inds, _) in tc_outs]
    x_real = jnp.concatenate(xr_slabs, axis=3)
    mask = jnp.concatenate([m for (_, _, m) in tc_outs], axis=3)
    return x_real, mask


def kernel(x_in, cl1_w, cl1_b, cl2_w, cl2_b, cl3_w, cl3_b,
           reg1_w, reg1_b, reg2_w, reg2_b, reg3_w, reg3_b):
    w2all = jnp.transpose(reg2_w, (0, 2, 1)).reshape(SUPER * 32, 256)
    w2r = w2all[:, 0:128]
    w2h = w2all[:, 128:256]
    b2t = reg2_b.T
    w3b = jnp.concatenate(
        [reg3_w[:, :, 0].T, reg3_b[:, 0].reshape(1, CLASSES)],
        axis=0).reshape(-1)
    x_real, mask = _run(x_in, cl1_w, cl1_b, cl2_w, cl2_b, cl3_w, cl3_b,
                        reg1_w, reg1_b, w2r, w2h, b2t, w3b)
    return (x_real, mask)


# speedup vs baseline: 2.0343x; 1.7316x over previous
import functools

import jax
import jax.numpy as jnp
from jax.experimental import pallas as pl
from jax.experimental.pallas import tpu as pltpu

CLASSES = 128
SUPER = 8
CF = CLASSES // SUPER
BW = 2048

_F32 = jnp.float32


def _lrelu(v):
    return jnp.maximum(v, 0.01 * v)


def _mm(w, v):
    return jax.lax.dot_general(w, v, (((1,), (0,)), ((), ())),
                               preferred_element_type=_F32)


def _body(x_ref, cl1_w_ref, cl1_b_ref, cl2_w_ref, cl2_b_ref, cl3_w_ref,
          cl3_b_ref, reg1_w_ref, reg1_b_ref, w2r_ref, w2h_ref, b2t_ref,
          we_ref, xreal_ref, mask_ref):
    x = x_ref[0, :, 0, :]

    h1 = _lrelu(_mm(cl1_w_ref[...], x) + cl1_b_ref[...].reshape(128, 1))
    h2 = _lrelu(_mm(cl2_w_ref[...], h1) + cl2_b_ref[...].reshape(128, 1))
    lg = _mm(cl3_w_ref[...], h2) + cl3_b_ref[...].reshape(CLASSES + 1, 1)
    mask_ref[0, 0, 0, :] = _lrelu(lg[CLASSES, :])

    cls = lg[0:CLASSES, :]
    m = jnp.max(cls, axis=0, keepdims=True)
    row_iota = jax.lax.broadcasted_iota(jnp.int32, (CLASSES, BW), 0)
    inds = jnp.min(jnp.where(cls == m, row_iota, CLASSES), axis=0,
                   keepdims=True)

    r1 = _lrelu(_mm(reg1_w_ref[...], x) + reg1_b_ref[...].reshape(128, 1))
    y = (_mm(w2r_ref[...], r1) +
         _mm(w2h_ref[...], h1))

    s = inds // CF
    oh8 = (jax.lax.broadcasted_iota(jnp.int32, (SUPER, BW), 0)
           == s).astype(_F32)
    b32 = _mm(b2t_ref[...], oh8)
    x32 = y[0:32, :]
    for e in range(1, SUPER):
        x32 = jnp.where(s == e, y[e * 32:(e + 1) * 32, :], x32)
    x32 = _lrelu(x32 + b32)

    oh = (row_iota == inds).astype(_F32)
    sel = _mm(we_ref[...], oh)
    reg = (jnp.sum(x32 * sel[0:32, :], axis=0, keepdims=True) +
           sel[32:33, :])
    xreal_ref[0, 0, 0, :] = ((inds.astype(_F32) + reg) *
                             (1.0 / float(CLASSES)))[0, :]


@jax.jit
def _run(x_in, cl1_w, cl1_b, cl2_w, cl2_b, cl3_w, cl3_b,
         reg1_w, reg1_b, w2r, w2h, b2t, we):
    B, C, H, W = x_in.shape
    grid = (B, W // BW)
    wspec = lambda shape: pl.BlockSpec(shape, lambda b, j: (0,) * len(shape))
    out_shapes = (
        jax.ShapeDtypeStruct((B, 1, 1, W), jnp.float32),
        jax.ShapeDtypeStruct((B, 1, 1, W), jnp.float32),
    )
    ospec = pl.BlockSpec((1, 1, 1, BW), lambda b, j: (b, 0, 0, j))
    return pl.pallas_call(
        _body,
        grid=grid,
        in_specs=[
            pl.BlockSpec((1, C, 1, BW), lambda b, j: (b, 0, 0, j)),
            wspec((128, 128)), wspec((128,)),
            wspec((128, 128)), wspec((128,)),
            wspec((CLASSES + 1, 128)), wspec((CLASSES + 1,)),
            wspec((128, 128)), wspec((128,)),
            wspec((256, 128)), wspec((256, 128)),
            wspec((32, SUPER)), wspec((33, 128)),
        ],
        out_specs=(ospec, ospec),
        out_shape=out_shapes,
    )(x_in, cl1_w, cl1_b, cl2_w, cl2_b, cl3_w, cl3_b,
      reg1_w, reg1_b, w2r, w2h, b2t, we)


def kernel(x_in, cl1_w, cl1_b, cl2_w, cl2_b, cl3_w, cl3_b,
           reg1_w, reg1_b, reg2_w, reg2_b, reg3_w, reg3_b):
    w2all = jnp.transpose(reg2_w, (0, 2, 1)).reshape(SUPER * 32, 256)
    w2r = w2all[:, 0:128]
    w2h = w2all[:, 128:256]
    b2t = reg2_b.T
    we = jnp.concatenate([reg3_w[:, :, 0].T,
                          reg3_b[:, 0].reshape(1, CLASSES)], axis=0)
    x_real, mask = _run(x_in, cl1_w, cl1_b, cl2_w, cl2_b, cl3_w, cl3_b,
                        reg1_w, reg1_b, w2r, w2h, b2t, we)
    return (x_real, mask)
